# baseline (speedup 1.0000x reference)
import functools

import jax
import jax.numpy as jnp
from jax import lax
from jax.experimental import pallas as pl
from jax.experimental.pallas import tpu as pltpu
from jax.experimental.pallas import tpu_sc as plsc

_N = 10000
_E = 160000
_H = 256
_HH = 128

_NB = 10
_BN = _N // _NB
_NBP = 5
_BNP = _N // _NBP
_EB = 80
_BE = _E // _EB

_NT = 16
_NHALF = 2
_EH = _E // _NHALF
_CH = 40
_EPT = _EH // _NT
_EBH = 40
_NCH = _EPT // _CH
_ACC_N = 10240
_RPT = _ACC_N // _NT
_RPT_LAST = _N - (_NT - 1) * _RPT

_F32 = jnp.float32


def _dotf(a, b):
    return jnp.dot(a, b, preferred_element_type=_F32)


def _embed_body(x_ref, w_ref, b_ref, o_ref):
    o_ref[...] = _dotf(x_ref[...], w_ref[...]) + b_ref[...]


def _tc_embed(x8, w8, b):
    return pl.pallas_call(
        _embed_body,
        grid=(_NB,),
        in_specs=[
            pl.BlockSpec((_BN, 8), lambda i: (i, 0)),
            pl.BlockSpec((8, _H), lambda i: (0, 0)),
            pl.BlockSpec((1, _H), lambda i: (0, 0)),
        ],
        out_specs=pl.BlockSpec((_BN, _H), lambda i: (i, 0)),
        out_shape=jax.ShapeDtypeStruct((_N, _H), _F32),
    )(x8, w8, b)


def _pack16(y, lo0):
    lo = jax.lax.bitcast_convert_type(y[:, lo0:lo0 + _HH], jnp.uint32)
    hi = jax.lax.bitcast_convert_type(y[:, lo0 + _HH:lo0 + 2 * _HH], jnp.uint32)
    rnd = jnp.uint32(0x8000)
    return ((hi + rnd) & jnp.uint32(0xFFFF0000)) | ((lo + rnd) >> 16)


def _unpack16(x):
    lo = jax.lax.bitcast_convert_type(x << 16, _F32)
    hi = jax.lax.bitcast_convert_type(x & jnp.uint32(0xFFFF0000), _F32)
    return lo, hi


def _p_body(h_ref, w_ref, a_ref, b_ref, c_ref):
    y = _dotf(h_ref[...], w_ref[...])
    a_ref[...] = _pack16(y, 0)
    b_ref[...] = _pack16(y, _H)
    c_ref[...] = y[:, 2 * _H:]


def _tc_p(h, wcat):
    outb = jax.ShapeDtypeStruct((_N, _HH), jnp.uint32)
    outf = jax.ShapeDtypeStruct((_N, _H), _F32)
    return pl.pallas_call(
        _p_body,
        grid=(_NBP,),
        in_specs=[
            pl.BlockSpec((_BNP, _H), lambda i: (i, 0)),
            pl.BlockSpec((_H, 3 * _H), lambda i: (0, 0)),
        ],
        out_specs=[
            pl.BlockSpec((_BNP, _HH), lambda i: (i, 0)),
            pl.BlockSpec((_BNP, _HH), lambda i: (i, 0)),
            pl.BlockSpec((_BNP, _H), lambda i: (i, 0)),
        ],
        out_shape=[outb, outb, outf],
    )(h, wcat)


def _edge_body(a_ref, b_ref, ea_ref, we_ref, b1_ref, w2_ref, b2_ref,
               m0_ref, m1_ref):
    alo, ahi = _unpack16(a_ref[...])
    blo, bhi = _unpack16(b_ref[...])
    cc = _dotf(ea_ref[...], we_ref[...]) + b1_ref[...]
    tlo = jnp.maximum(alo + blo + cc[:, :_HH], 0.0)
    thi = jnp.maximum(ahi + bhi + cc[:, _HH:], 0.0)
    t = jnp.concatenate([tlo, thi], axis=1).astype(jnp.bfloat16)
    m = _dotf(t, w2_ref[...]) + b2_ref[...]
    m0_ref[...] = m[:, :_HH]
    m1_ref[...] = m[:, _HH:]


def _tc_edge(asrc, bdst, ea, we_ea, b1, w2, b2, half):
    out = jax.ShapeDtypeStruct((_EH, _HH), _F32)
    ebo = half * _EBH
    return pl.pallas_call(
        _edge_body,
        grid=(_EBH,),
        in_specs=[
            pl.BlockSpec((_BE, _HH), lambda i: (i, 0)),
            pl.BlockSpec((_BE, _HH), lambda i: (i, 0)),
            pl.BlockSpec((_BE, 16), lambda i: (i + ebo, 0)),
            pl.BlockSpec((16, _H), lambda i: (0, 0)),
            pl.BlockSpec((1, _H), lambda i: (0, 0)),
            pl.BlockSpec((_H, _H), lambda i: (0, 0)),
            pl.BlockSpec((1, _H), lambda i: (0, 0)),
        ],
        out_specs=[pl.BlockSpec((_BE, _HH), lambda i: (i, 0))] * 2,
        out_shape=[out, out],
    )(asrc, bdst, ea, we_ea, b1, w2, b2)


def _node_body(h_ref, hw_ref, ma0_ref, ma1_ref, mb0_ref, mb1_ref,
               w1t_ref, w1b_ref, b1_ref, w2_ref, b2_ref, o_ref):
    m0 = ma0_ref[...] + mb0_ref[...]
    m1 = ma1_ref[...] + mb1_ref[...]
    t = (hw_ref[...] + _dotf(m0, w1t_ref[...])
         + _dotf(m1, w1b_ref[...]) + b1_ref[...])
    t = jnp.maximum(t, 0.0)
    o_ref[...] = h_ref[...] + _dotf(t, w2_ref[...]) + b2_ref[...]


def _tc_node(h, hw, ma0, ma1, mb0, mb1, w1t, w1b, b1, w2, b2):
    return pl.pallas_call(
        _node_body,
        grid=(_NB,),
        in_specs=[
            pl.BlockSpec((_BN, _H), lambda i: (i, 0)),
            pl.BlockSpec((_BN, _H), lambda i: (i, 0)),
            pl.BlockSpec((_BN, _HH), lambda i: (i, 0)),
            pl.BlockSpec((_BN, _HH), lambda i: (i, 0)),
            pl.BlockSpec((_BN, _HH), lambda i: (i, 0)),
            pl.BlockSpec((_BN, _HH), lambda i: (i, 0)),
            pl.BlockSpec((_HH, _H), lambda i: (0, 0)),
            pl.BlockSpec((_HH, _H), lambda i: (0, 0)),
            pl.BlockSpec((1, _H), lambda i: (0, 0)),
            pl.BlockSpec((_H, _H), lambda i: (0, 0)),
            pl.BlockSpec((1, _H), lambda i: (0, 0)),
        ],
        out_specs=pl.BlockSpec((_BN, _H), lambda i: (i, 0)),
        out_shape=jax.ShapeDtypeStruct((_N, _H), _F32),
    )(h, hw, ma0, ma1, mb0, mb1, w1t, w1b, b1, w2, b2)


def _final_body(h_ref, u_ref, wph_ref, wpu_ref, b1_ref, w2_ref, b2_ref,
                w3_ref, b3_ref, o_ref, acc_ref):
    i = pl.program_id(0)

    @pl.when(i == 0)
    def _():
        acc_ref[...] = jnp.zeros_like(acc_ref)

    acc_ref[...] += jnp.sum(h_ref[...], axis=0, keepdims=True)

    @pl.when(i == pl.num_programs(0) - 1)
    def _():
        gr = acc_ref[...] * (1.0 / _N)
        z = jnp.maximum(_dotf(gr, wph_ref[...]) + _dotf(u_ref[...], wpu_ref[...])
                        + b1_ref[...], 0.0)
        z = jnp.maximum(_dotf(z, w2_ref[...]) + b2_ref[...], 0.0)
        o_ref[...] = _dotf(z, w3_ref[...]) + b3_ref[...]


def _tc_final(h, u, wph, wpu, b1, w2, b2, w3, b3):
    return pl.pallas_call(
        _final_body,
        grid=(_NB,),
        in_specs=[
            pl.BlockSpec((_BN, _H), lambda i: (i, 0)),
            pl.BlockSpec((1, 4), lambda i: (0, 0)),
            pl.BlockSpec((_H, _H), lambda i: (0, 0)),
            pl.BlockSpec((4, _H), lambda i: (0, 0)),
            pl.BlockSpec((1, _H), lambda i: (0, 0)),
            pl.BlockSpec((_H, _HH), lambda i: (0, 0)),
            pl.BlockSpec((1, _HH), lambda i: (0, 0)),
            pl.BlockSpec((_HH, 3), lambda i: (0, 0)),
            pl.BlockSpec((1, 3), lambda i: (0, 0)),
        ],
        out_specs=pl.BlockSpec((1, 3), lambda i: (0, 0)),
        out_shape=jax.ShapeDtypeStruct((1, 3), _F32),
        scratch_shapes=[pltpu.VMEM((1, _H), _F32)],
    )(h, u, wph, wpu, b1, w2, b2, w3, b3)


def _gather_body(off, a_hbm, b_hbm, src_hbm, dst_hbm, ao_hbm, bo_hbm,
                 idx_v, rows0, rows1, sg0, sg1, sw0, sw1):
    c = lax.axis_index("c")
    s = lax.axis_index("s")
    base = s * _EPT

    def run(tab, idxs, out):
        pltpu.sync_copy(idxs.at[pl.ds(pl.multiple_of(off + s * _EPT, 8),
                                      _EPT)], idx_v)

        def g_desc(i, buf, sem):
            io = pl.multiple_of(i * _CH, 8)
            return pltpu.make_async_copy(
                tab.at[idx_v.at[pl.ds(io, _CH)]], buf, sem)

        def w_desc(i, buf, sem):
            off = pl.multiple_of(base + i * _CH, 8)
            return pltpu.make_async_copy(buf, out.at[pl.ds(off, _CH)], sem)

        g_desc(0, rows0, sg0).start()

        def body(j, carry):
            i0 = 2 * j
            i1 = i0 + 1

            @pl.when(j > 0)
            def _():
                w_desc(i0 - 1, rows1, sw1).wait()

            g_desc(i0, rows0, sg0).wait()
            g_desc(i1, rows1, sg1).start()
            w_desc(i0, rows0, sw0).start()
            g_desc(i1, rows1, sg1).wait()
            w_desc(i0, rows0, sw0).wait()
            w_desc(i1, rows1, sw1).start()
            g_desc(i0 + 2, rows0, sg0).start()
            return carry

        lax.fori_loop(0, (_NCH - 1) // 2, body, 0)
        w_desc(_NCH - 2, rows1, sw1).wait()
        g_desc(_NCH - 1, rows0, sg0).wait()
        w_desc(_NCH - 1, rows0, sw0).start()
        w_desc(_NCH - 1, rows0, sw0).wait()

    @pl.when(c == 0)
    def _():
        run(a_hbm, src_hbm, ao_hbm)

    @pl.when(c == 1)
    def _():
        run(b_hbm, dst_hbm, bo_hbm)


def _sc_gather(a, b, src, dst, half):
    mesh = plsc.VectorSubcoreMesh(core_axis_name="c", subcore_axis_name="s")
    out = jax.ShapeDtypeStruct((_EH, _HH), jnp.uint32)
    f = functools.partial(
        pl.kernel,
        mesh=mesh,
        out_type=(out, out),
        scratch_types=[
            pltpu.VMEM((_EPT,), jnp.int32),
            pltpu.VMEM((_CH, _HH), jnp.uint32),
            pltpu.VMEM((_CH, _HH), jnp.uint32),
            pltpu.SemaphoreType.DMA,
            pltpu.SemaphoreType.DMA,
            pltpu.SemaphoreType.DMA,
            pltpu.SemaphoreType.DMA,
        ],
    )(functools.partial(_gather_body, half * _EH))
    return f(a, b, src, dst)


def _scatter_body(off, m0_hbm, m1_hbm, dst_hbm, zero_hbm, o0_hbm, o1_hbm,
                  idx0, idx1, buf0, buf1, sl0, sl1, acc_sh):
    c = lax.axis_index("c")
    s = lax.axis_index("s")
    r0 = pl.multiple_of(s * _RPT, 8)
    pltpu.sync_copy(zero_hbm, acc_sh.at[pl.ds(r0, _RPT)])
    plsc.subcore_barrier()

    base = s * _EPT

    def run(m_hbm):
        def l_descs(i, idxb, buf, sem):
            mo = pl.multiple_of(base + i * _CH, 8)
            do = pl.multiple_of(off + base + i * _CH, 8)
            return (pltpu.make_async_copy(dst_hbm.at[pl.ds(do, _CH)],
                                          idxb, sem),
                    pltpu.make_async_copy(m_hbm.at[pl.ds(mo, _CH)],
                                          buf, sem))

        def start(i, idxb, buf, sem):
            di, dm = l_descs(i, idxb, buf, sem)
            di.start()
            dm.start()

        def wait(i, idxb, buf, sem):
            di, dm = l_descs(i, idxb, buf, sem)
            di.wait()
            dm.wait()

        def scat(idxb, buf):
            pltpu.sync_copy(buf, acc_sh.at[idxb], add=True)

        start(0, idx0, buf0, sl0)

        def body(j, carry):
            i0 = 2 * j
            i1 = i0 + 1
            wait(i0, idx0, buf0, sl0)
            start(i1, idx1, buf1, sl1)
            scat(idx0, buf0)
            wait(i1, idx1, buf1, sl1)

            @pl.when(j < (_NCH - 1) // 2 - 1)
            def _():
                start(i0 + 2, idx0, buf0, sl0)

            scat(idx1, buf1)
            return carry

        lax.fori_loop(0, (_NCH - 1) // 2, body, 0)
        start(_NCH - 1, idx0, buf0, sl0)
        wait(_NCH - 1, idx0, buf0, sl0)
        scat(idx0, buf0)

    @pl.when(c == 0)
    def _():
        run(m0_hbm)

    @pl.when(c == 1)
    def _():
        run(m1_hbm)

    plsc.subcore_barrier()

    @pl.when(jnp.logical_and(c == 0, s < _NT - 1))
    def _():
        pltpu.sync_copy(acc_sh.at[pl.ds(r0, _RPT)], o0_hbm.at[pl.ds(r0, _RPT)])

    @pl.when(jnp.logical_and(c == 1, s < _NT - 1))
    def _():
        pltpu.sync_copy(acc_sh.at[pl.ds(r0, _RPT)], o1_hbm.at[pl.ds(r0, _RPT)])

    @pl.when(jnp.logical_and(c == 0, s == _NT - 1))
    def _():
        pltpu.sync_copy(acc_sh.at[pl.ds(r0, _RPT_LAST)],
                        o0_hbm.at[pl.ds(r0, _RPT_LAST)])

    @pl.when(jnp.logical_and(c == 1, s == _NT - 1))
    def _():
        pltpu.sync_copy(acc_sh.at[pl.ds(r0, _RPT_LAST)],
                        o1_hbm.at[pl.ds(r0, _RPT_LAST)])


def _sc_scatter(m0, m1, dst, zeros_half, half):
    mesh = plsc.VectorSubcoreMesh(core_axis_name="c", subcore_axis_name="s")
    out = jax.ShapeDtypeStruct((_N, _HH), _F32)
    f = functools.partial(
        pl.kernel,
        mesh=mesh,
        out_type=(out, out),
        scratch_types=[
            pltpu.VMEM((_CH,), jnp.int32),
            pltpu.VMEM((_CH,), jnp.int32),
            pltpu.VMEM((_CH, _HH), _F32),
            pltpu.VMEM((_CH, _HH), _F32),
            pltpu.SemaphoreType.DMA,
            pltpu.SemaphoreType.DMA,
            pltpu.VMEM_SHARED((_ACC_N, _HH), _F32),
        ],
    )(functools.partial(_scatter_body, half * _EH))
    return f(m0, m1, dst, zeros_half)


def kernel(x, edge_index, edge_attr, u, batch, W_embed, b_embed, Wrel, brel,
           Wroot, We1, be1, We2, be2, Wn1, bn1, Wn2, bn2, Wp1, bp1, Wp2, bp2,
           Wp3, bp3):
    src = edge_index[0]
    dst = edge_index[1]

    x8 = jnp.pad(x, ((0, 0), (0, 1)))
    w8 = jnp.pad(W_embed, ((0, 1), (0, 0)))
    h = _tc_embed(x8, w8, b_embed.reshape(1, _H))

    zeros_half = jnp.zeros((_RPT, _HH), _F32)
    ea16 = edge_attr.astype(jnp.bfloat16)

    for i in range(4):
        wcat = jnp.concatenate(
            [We1[i][:_H], We1[i][_H:2 * _H], Wn1[i][:_H]], axis=1)
        a, b, hw = _tc_p(h, wcat)
        we_ea = We1[i][2 * _H:].astype(jnp.bfloat16)
        w2b = We2[i].astype(jnp.bfloat16)
        b1r = be1[i].reshape(1, _H)
        b2r = be2[i].reshape(1, _H)
        aA, bA = _sc_gather(a, b, src, dst, 0)
        aB, bB = _sc_gather(a, b, src, dst, 1)
        mA0, mA1 = _tc_edge(aA, bA, ea16, we_ea, b1r, w2b, b2r, 0)
        mB0, mB1 = _tc_edge(aB, bB, ea16, we_ea, b1r, w2b, b2r, 1)
        gA0, gA1 = _sc_scatter(mA0, mA1, dst, zeros_half, 0)
        gB0, gB1 = _sc_scatter(mB0, mB1, dst, zeros_half, 1)
        h = _tc_node(h, hw, gA0, gA1, gB0, gB1,
                     Wn1[i][_H:_H + _HH], Wn1[i][_H + _HH:],
                     bn1[i].reshape(1, _H), Wn2[i], bn2[i].reshape(1, _H))

    return _tc_final(h, u, Wp1[:_H], Wp1[_H:], bp1.reshape(1, _H),
                     Wp2, bp2.reshape(1, _HH), Wp3, bp3.reshape(1, 3))


# speedup vs baseline: 1.3569x; 1.3569x over previous
import functools

import jax
import jax.numpy as jnp
from jax import lax
from jax.experimental import pallas as pl
from jax.experimental.pallas import tpu as pltpu
from jax.experimental.pallas import tpu_sc as plsc

_N = 10000
_E = 160000
_H = 256
_HH = 128

_NB = 10
_BN = _N // _NB
_NBP = 5
_BNP = _N // _NBP
_EB = 80
_BE = _E // _EB

_NT = 16
_CH = 128
_EPT = _E // _NT
_NCHF = _EPT // _CH
_CHT = _EPT - _NCHF * _CH
_ACC_N = 10240
_RPT = _ACC_N // _NT
_RPT_LAST = _N - (_NT - 1) * _RPT

_F32 = jnp.float32


def _dotf(a, b):
    return jnp.dot(a, b, preferred_element_type=_F32)


def _embed_body(x_ref, w_ref, b_ref, o_ref):
    o_ref[...] = _dotf(x_ref[...], w_ref[...]) + b_ref[...]


def _tc_embed(x8, w8, b):
    return pl.pallas_call(
        _embed_body,
        grid=(_NB,),
        in_specs=[
            pl.BlockSpec((_BN, 8), lambda i: (i, 0)),
            pl.BlockSpec((8, _H), lambda i: (0, 0)),
            pl.BlockSpec((1, _H), lambda i: (0, 0)),
        ],
        out_specs=pl.BlockSpec((_BN, _H), lambda i: (i, 0)),
        out_shape=jax.ShapeDtypeStruct((_N, _H), _F32),
    )(x8, w8, b)


def _pack16(y, lo0):
    lo = jax.lax.bitcast_convert_type(y[:, lo0:lo0 + _HH], jnp.uint32)
    hi = jax.lax.bitcast_convert_type(y[:, lo0 + _HH:lo0 + 2 * _HH], jnp.uint32)
    rnd = jnp.uint32(0x8000)
    return ((hi + rnd) & jnp.uint32(0xFFFF0000)) | ((lo + rnd) >> 16)


def _unpack16(x):
    lo = jax.lax.bitcast_convert_type(x << 16, _F32)
    hi = jax.lax.bitcast_convert_type(x & jnp.uint32(0xFFFF0000), _F32)
    return lo, hi


def _p_body(h_ref, w_ref, a_ref, b_ref, c_ref):
    y = _dotf(h_ref[...], w_ref[...])
    a_ref[...] = _pack16(y, 0)
    b_ref[...] = _pack16(y, _H)
    c_ref[...] = y[:, 2 * _H:]


def _tc_p(h, wcat):
    outb = jax.ShapeDtypeStruct((_N, _HH), jnp.uint32)
    outf = jax.ShapeDtypeStruct((_N, _H), _F32)
    return pl.pallas_call(
        _p_body,
        grid=(_NBP,),
        in_specs=[
            pl.BlockSpec((_BNP, _H), lambda i: (i, 0)),
            pl.BlockSpec((_H, 3 * _H), lambda i: (0, 0)),
        ],
        out_specs=[
            pl.BlockSpec((_BNP, _HH), lambda i: (i, 0)),
            pl.BlockSpec((_BNP, _HH), lambda i: (i, 0)),
            pl.BlockSpec((_BNP, _H), lambda i: (i, 0)),
        ],
        out_shape=[outb, outb, outf],
    )(h, wcat)


def _edge_body(a_ref, b_ref, ea_ref, we_ref, b1_ref, w2_ref, b2_ref,
               m0_ref, m1_ref):
    alo, ahi = _unpack16(a_ref[...])
    blo, bhi = _unpack16(b_ref[...])
    cc = _dotf(ea_ref[...], we_ref[...]) + b1_ref[...]
    tlo = jnp.maximum(alo + blo + cc[:, :_HH], 0.0)
    thi = jnp.maximum(ahi + bhi + cc[:, _HH:], 0.0)
    t = jnp.concatenate([tlo, thi], axis=1).astype(jnp.bfloat16)
    m = _dotf(t, w2_ref[...]) + b2_ref[...]
    m0_ref[...] = m[:, :_HH]
    m1_ref[...] = m[:, _HH:]


def _tc_edge(asrc, bdst, ea, we_ea, b1, w2, b2):
    out = jax.ShapeDtypeStruct((_E, _HH), _F32)
    return pl.pallas_call(
        _edge_body,
        grid=(_EB,),
        in_specs=[
            pl.BlockSpec((_BE, _HH), lambda i: (i, 0)),
            pl.BlockSpec((_BE, _HH), lambda i: (i, 0)),
            pl.BlockSpec((_BE, 16), lambda i: (i, 0)),
            pl.BlockSpec((16, _H), lambda i: (0, 0)),
            pl.BlockSpec((1, _H), lambda i: (0, 0)),
            pl.BlockSpec((_H, _H), lambda i: (0, 0)),
            pl.BlockSpec((1, _H), lambda i: (0, 0)),
        ],
        out_specs=[pl.BlockSpec((_BE, _HH), lambda i: (i, 0))] * 2,
        out_shape=[out, out],
    )(asrc, bdst, ea, we_ea, b1, w2, b2)


def _node_body(h_ref, hw_ref, m0_ref, m1_ref, w1t_ref, w1b_ref, b1_ref,
               w2_ref, b2_ref, o_ref):
    t = (hw_ref[...] + _dotf(m0_ref[...], w1t_ref[...])
         + _dotf(m1_ref[...], w1b_ref[...]) + b1_ref[...])
    t = jnp.maximum(t, 0.0)
    o_ref[...] = h_ref[...] + _dotf(t, w2_ref[...]) + b2_ref[...]


def _tc_node(h, hw, m0, m1, w1t, w1b, b1, w2, b2):
    return pl.pallas_call(
        _node_body,
        grid=(_NB,),
        in_specs=[
            pl.BlockSpec((_BN, _H), lambda i: (i, 0)),
            pl.BlockSpec((_BN, _H), lambda i: (i, 0)),
            pl.BlockSpec((_BN, _HH), lambda i: (i, 0)),
            pl.BlockSpec((_BN, _HH), lambda i: (i, 0)),
            pl.BlockSpec((_HH, _H), lambda i: (0, 0)),
            pl.BlockSpec((_HH, _H), lambda i: (0, 0)),
            pl.BlockSpec((1, _H), lambda i: (0, 0)),
            pl.BlockSpec((_H, _H), lambda i: (0, 0)),
            pl.BlockSpec((1, _H), lambda i: (0, 0)),
        ],
        out_specs=pl.BlockSpec((_BN, _H), lambda i: (i, 0)),
        out_shape=jax.ShapeDtypeStruct((_N, _H), _F32),
    )(h, hw, m0, m1, w1t, w1b, b1, w2, b2)


def _final_body(h_ref, u_ref, wph_ref, wpu_ref, b1_ref, w2_ref, b2_ref,
                w3_ref, b3_ref, o_ref, acc_ref):
    i = pl.program_id(0)

    @pl.when(i == 0)
    def _():
        acc_ref[...] = jnp.zeros_like(acc_ref)

    acc_ref[...] += jnp.sum(h_ref[...], axis=0, keepdims=True)

    @pl.when(i == pl.num_programs(0) - 1)
    def _():
        gr = acc_ref[...] * (1.0 / _N)
        z = jnp.maximum(_dotf(gr, wph_ref[...]) + _dotf(u_ref[...], wpu_ref[...])
                        + b1_ref[...], 0.0)
        z = jnp.maximum(_dotf(z, w2_ref[...]) + b2_ref[...], 0.0)
        o_ref[...] = _dotf(z, w3_ref[...]) + b3_ref[...]


def _tc_final(h, u, wph, wpu, b1, w2, b2, w3, b3):
    return pl.pallas_call(
        _final_body,
        grid=(_NB,),
        in_specs=[
            pl.BlockSpec((_BN, _H), lambda i: (i, 0)),
            pl.BlockSpec((1, 4), lambda i: (0, 0)),
            pl.BlockSpec((_H, _H), lambda i: (0, 0)),
            pl.BlockSpec((4, _H), lambda i: (0, 0)),
            pl.BlockSpec((1, _H), lambda i: (0, 0)),
            pl.BlockSpec((_H, _HH), lambda i: (0, 0)),
            pl.BlockSpec((1, _HH), lambda i: (0, 0)),
            pl.BlockSpec((_HH, 3), lambda i: (0, 0)),
            pl.BlockSpec((1, 3), lambda i: (0, 0)),
        ],
        out_specs=pl.BlockSpec((1, 3), lambda i: (0, 0)),
        out_shape=jax.ShapeDtypeStruct((1, 3), _F32),
        scratch_shapes=[pltpu.VMEM((1, _H), _F32)],
    )(h, u, wph, wpu, b1, w2, b2, w3, b3)


def _pipeline(n_full, g_desc, w_desc, r0, sg0, sw0, r1, sg1, sw1):
    g_desc(0, r0, sg0).start()

    def body(j, carry):
        i0 = 2 * j
        i1 = i0 + 1

        @pl.when(j > 0)
        def _():
            w_desc(i0 - 1, r1, sw1).wait()

        g_desc(i0, r0, sg0).wait()
        g_desc(i1, r1, sg1).start()
        w_desc(i0, r0, sw0).start()
        g_desc(i1, r1, sg1).wait()
        w_desc(i0, r0, sw0).wait()

        @pl.when(i0 + 2 < n_full)
        def _():
            g_desc(i0 + 2, r0, sg0).start()

        w_desc(i1, r1, sw1).start()
        return carry

    lax.fori_loop(0, n_full // 2, body, 0)
    w_desc(n_full - 1, r1, sw1).wait()


def _gather_body(a_hbm, b_hbm, src_hbm, dst_hbm, ao_hbm, bo_hbm,
                 idx_v, rows0, rows1, sg0, sg1, sw0, sw1):
    c = lax.axis_index("c")
    s = lax.axis_index("s")
    base = s * _EPT

    def run(tab, idxs, out):
        pltpu.sync_copy(idxs.at[pl.ds(pl.multiple_of(base, 8), _EPT)], idx_v)

        def g_desc(i, buf, sem, n=_CH):
            io = pl.multiple_of(i * _CH, 8)
            return pltpu.make_async_copy(
                tab.at[idx_v.at[pl.ds(io, n)]], buf, sem)

        def w_desc(i, buf, sem, n=_CH):
            off = pl.multiple_of(base + i * _CH, 8)
            return pltpu.make_async_copy(buf, out.at[pl.ds(off, n)], sem)

        _pipeline(_NCHF, g_desc, w_desc, rows0, sg0, sw0, rows1, sg1, sw1)

        tl = rows0.at[pl.ds(0, _CHT)]
        g_desc(_NCHF, tl, sg0, _CHT).start()
        g_desc(_NCHF, tl, sg0, _CHT).wait()
        w_desc(_NCHF, tl, sw0, _CHT).start()
        w_desc(_NCHF, tl, sw0, _CHT).wait()

    @pl.when(c == 0)
    def _():
        run(a_hbm, src_hbm, ao_hbm)

    @pl.when(c == 1)
    def _():
        run(b_hbm, dst_hbm, bo_hbm)


def _sc_gather(a, b, src, dst):
    mesh = plsc.VectorSubcoreMesh(core_axis_name="c", subcore_axis_name="s")
    out = jax.ShapeDtypeStruct((_E, _HH), jnp.uint32)
    f = functools.partial(
        pl.kernel,
        mesh=mesh,
        out_type=(out, out),
        scratch_types=[
            pltpu.VMEM((_EPT,), jnp.int32),
            pltpu.VMEM((_CH, _HH), jnp.uint32),
            pltpu.VMEM((_CH, _HH), jnp.uint32),
            pltpu.SemaphoreType.DMA,
            pltpu.SemaphoreType.DMA,
            pltpu.SemaphoreType.DMA,
            pltpu.SemaphoreType.DMA,
        ],
    )(_gather_body)
    return f(a, b, src, dst)


def _scatter_body(m0_hbm, m1_hbm, dst_hbm, zero_hbm, o0_hbm, o1_hbm,
                  idx0, idx1, idxt, buf0, buf1, buft, sl0, sl1, acc_sh):
    c = lax.axis_index("c")
    s = lax.axis_index("s")
    r0 = pl.multiple_of(s * _RPT, 8)
    pltpu.sync_copy(zero_hbm, acc_sh.at[pl.ds(r0, _RPT)])
    plsc.subcore_barrier()

    base = s * _EPT

    def run(m_hbm):
        def descs(i, bufs, sem, n=_CH):
            idxb, buf = bufs
            off = pl.multiple_of(base + i * _CH, 8)
            return (pltpu.make_async_copy(dst_hbm.at[pl.ds(off, n)],
                                          idxb, sem),
                    pltpu.make_async_copy(m_hbm.at[pl.ds(off, n)], buf, sem))

        class _Loads:
            def __init__(self, i, bufs, sem, n=_CH):
                self.d = descs(i, bufs, sem, n)

            def start(self):
                for d in self.d:
                    d.start()

            def wait(self):
                for d in self.d:
                    d.wait()

        class _Scat:
            def __init__(self, i, bufs, sem, n=_CH):
                self.bufs = bufs

            def start(self):
                idxb, buf = self.bufs
                pltpu.sync_copy(buf, acc_sh.at[idxb], add=True)

            def wait(self):
                pass

        _pipeline(_NCHF,
                  lambda i, bufs, sem, n=_CH: _Loads(i, bufs, sem, n),
                  lambda i, bufs, sem, n=_CH: _Scat(i, bufs, sem, n),
                  (idx0, buf0), sl0, sl0, (idx1, buf1), sl1, sl1)

        tb = (idxt, buft)
        _Loads(_NCHF, tb, sl0, _CHT).start()
        _Loads(_NCHF, tb, sl0, _CHT).wait()
        _Scat(_NCHF, tb, sl0, _CHT).start()

    @pl.when(c == 0)
    def _():
        run(m0_hbm)

    @pl.when(c == 1)
    def _():
        run(m1_hbm)

    plsc.subcore_barrier()

    @pl.when(jnp.logical_and(c == 0, s < _NT - 1))
    def _():
        pltpu.sync_copy(acc_sh.at[pl.ds(r0, _RPT)], o0_hbm.at[pl.ds(r0, _RPT)])

    @pl.when(jnp.logical_and(c == 1, s < _NT - 1))
    def _():
        pltpu.sync_copy(acc_sh.at[pl.ds(r0, _RPT)], o1_hbm.at[pl.ds(r0, _RPT)])

    @pl.when(jnp.logical_and(c == 0, s == _NT - 1))
    def _():
        pltpu.sync_copy(acc_sh.at[pl.ds(r0, _RPT_LAST)],
                        o0_hbm.at[pl.ds(r0, _RPT_LAST)])

    @pl.when(jnp.logical_and(c == 1, s == _NT - 1))
    def _():
        pltpu.sync_copy(acc_sh.at[pl.ds(r0, _RPT_LAST)],
                        o1_hbm.at[pl.ds(r0, _RPT_LAST)])


def _sc_scatter(m0, m1, dst, zeros_half):
    mesh = plsc.VectorSubcoreMesh(core_axis_name="c", subcore_axis_name="s")
    out = jax.ShapeDtypeStruct((_N, _HH), _F32)
    f = functools.partial(
        pl.kernel,
        mesh=mesh,
        out_type=(out, out),
        scratch_types=[
            pltpu.VMEM((_CH,), jnp.int32),
            pltpu.VMEM((_CH,), jnp.int32),
            pltpu.VMEM((_CHT,), jnp.int32),
            pltpu.VMEM((_CH, _HH), _F32),
            pltpu.VMEM((_CH, _HH), _F32),
            pltpu.VMEM((_CHT, _HH), _F32),
            pltpu.SemaphoreType.DMA,
            pltpu.SemaphoreType.DMA,
            pltpu.VMEM_SHARED((_ACC_N, _HH), _F32),
        ],
    )(_scatter_body)
    return f(m0, m1, dst, zeros_half)


def kernel(x, edge_index, edge_attr, u, batch, W_embed, b_embed, Wrel, brel,
           Wroot, We1, be1, We2, be2, Wn1, bn1, Wn2, bn2, Wp1, bp1, Wp2, bp2,
           Wp3, bp3):
    src = edge_index[0]
    dst = edge_index[1]

    x8 = jnp.pad(x, ((0, 0), (0, 1)))
    w8 = jnp.pad(W_embed, ((0, 1), (0, 0)))
    h = _tc_embed(x8, w8, b_embed.reshape(1, _H))

    zeros_half = jnp.zeros((_RPT, _HH), _F32)
    ea16 = edge_attr.astype(jnp.bfloat16)

    for i in range(4):
        wcat = jnp.concatenate(
            [We1[i][:_H], We1[i][_H:2 * _H], Wn1[i][:_H]], axis=1)
        a, b, hw = _tc_p(h, wcat)
        asrc, bdst = _sc_gather(a, b, src, dst)
        m0, m1 = _tc_edge(asrc, bdst, ea16,
                          We1[i][2 * _H:].astype(jnp.bfloat16),
                          be1[i].reshape(1, _H), We2[i].astype(jnp.bfloat16),
                          be2[i].reshape(1, _H))
        g0, g1 = _sc_scatter(m0, m1, dst, zeros_half)
        h = _tc_node(h, hw, g0, g1, Wn1[i][_H:_H + _HH], Wn1[i][_H + _HH:],
                     bn1[i].reshape(1, _H), Wn2[i], bn2[i].reshape(1, _H))

    return _tc_final(h, u, Wp1[:_H], Wp1[_H:], bp1.reshape(1, _H),
                     Wp2, bp2.reshape(1, _HH), Wp3, bp3.reshape(1, 3))


# speedup vs baseline: 1.3799x; 1.0170x over previous
import functools

import jax
import jax.numpy as jnp
from jax import lax
from jax.experimental import pallas as pl
from jax.experimental.pallas import tpu as pltpu
from jax.experimental.pallas import tpu_sc as plsc

_N = 10000
_E = 160000
_H = 256
_HH = 128

_NB = 10
_BN = _N // _NB
_NBP = 5
_BNP = _N // _NBP
_EB = 80
_BE = _E // _EB

_NT = 16
_CH = 128
_EPT = _E // _NT
_NCHF = _EPT // _CH
_CHT = _EPT - _NCHF * _CH
_ACC_N = 10240
_RPT = _ACC_N // _NT
_RPT_LAST = _N - (_NT - 1) * _RPT

_F32 = jnp.float32


def _dotf(a, b):
    return jnp.dot(a, b, preferred_element_type=_F32)


def _embed_body(x_ref, w_ref, b_ref, wc_ref, h_ref, a_ref, b2_ref, c_ref):
    h = _dotf(x_ref[...], w_ref[...]) + b_ref[...]
    h_ref[...] = h
    y = _dotf(h, wc_ref[...])
    a_ref[...] = _pack16(y, 0)
    b2_ref[...] = _pack16(y, _H)
    c_ref[...] = y[:, 2 * _H:]


def _tc_embed_p(x8, w8, b, wcat):
    outb = jax.ShapeDtypeStruct((_N, _HH), jnp.uint32)
    outf = jax.ShapeDtypeStruct((_N, _H), _F32)
    return pl.pallas_call(
        _embed_body,
        grid=(_NBP,),
        in_specs=[
            pl.BlockSpec((_BNP, 8), lambda i: (i, 0)),
            pl.BlockSpec((8, _H), lambda i: (0, 0)),
            pl.BlockSpec((1, _H), lambda i: (0, 0)),
            pl.BlockSpec((_H, 3 * _H), lambda i: (0, 0)),
        ],
        out_specs=[
            pl.BlockSpec((_BNP, _H), lambda i: (i, 0)),
            pl.BlockSpec((_BNP, _HH), lambda i: (i, 0)),
            pl.BlockSpec((_BNP, _HH), lambda i: (i, 0)),
            pl.BlockSpec((_BNP, _H), lambda i: (i, 0)),
        ],
        out_shape=[outf, outb, outb, outf],
    )(x8, w8, b, wcat)


def _pack16(y, lo0):
    lo = jax.lax.bitcast_convert_type(y[:, lo0:lo0 + _HH], jnp.uint32)
    hi = jax.lax.bitcast_convert_type(y[:, lo0 + _HH:lo0 + 2 * _HH], jnp.uint32)
    rnd = jnp.uint32(0x8000)
    return ((hi + rnd) & jnp.uint32(0xFFFF0000)) | ((lo + rnd) >> 16)


def _unpack16(x):
    lo = jax.lax.bitcast_convert_type(x << 16, _F32)
    hi = jax.lax.bitcast_convert_type(x & jnp.uint32(0xFFFF0000), _F32)
    return lo, hi


def _p_body(h_ref, w_ref, a_ref, b_ref, c_ref):
    y = _dotf(h_ref[...], w_ref[...])
    a_ref[...] = _pack16(y, 0)
    b_ref[...] = _pack16(y, _H)
    c_ref[...] = y[:, 2 * _H:]


def _tc_p(h, wcat):
    outb = jax.ShapeDtypeStruct((_N, _HH), jnp.uint32)
    outf = jax.ShapeDtypeStruct((_N, _H), _F32)
    return pl.pallas_call(
        _p_body,
        grid=(_NBP,),
        in_specs=[
            pl.BlockSpec((_BNP, _H), lambda i: (i, 0)),
            pl.BlockSpec((_H, 3 * _H), lambda i: (0, 0)),
        ],
        out_specs=[
            pl.BlockSpec((_BNP, _HH), lambda i: (i, 0)),
            pl.BlockSpec((_BNP, _HH), lambda i: (i, 0)),
            pl.BlockSpec((_BNP, _H), lambda i: (i, 0)),
        ],
        out_shape=[outb, outb, outf],
    )(h, wcat)


def _edge_body(a_ref, b_ref, ea_ref, we_ref, b1_ref, w2_ref, b2_ref,
               m0_ref, m1_ref):
    alo, ahi = _unpack16(a_ref[...])
    blo, bhi = _unpack16(b_ref[...])
    cc = _dotf(ea_ref[...], we_ref[...]) + b1_ref[...]
    tlo = jnp.maximum(alo + blo + cc[:, :_HH], 0.0)
    thi = jnp.maximum(ahi + bhi + cc[:, _HH:], 0.0)
    t = jnp.concatenate([tlo, thi], axis=1).astype(jnp.bfloat16)
    m = _dotf(t, w2_ref[...]) + b2_ref[...]
    m0_ref[...] = m[:, :_HH]
    m1_ref[...] = m[:, _HH:]


def _tc_edge(asrc, bdst, ea, we_ea, b1, w2, b2):
    out = jax.ShapeDtypeStruct((_E, _HH), _F32)
    return pl.pallas_call(
        _edge_body,
        grid=(_EB,),
        in_specs=[
            pl.BlockSpec((_BE, _HH), lambda i: (i, 0)),
            pl.BlockSpec((_BE, _HH), lambda i: (i, 0)),
            pl.BlockSpec((_BE, 16), lambda i: (i, 0)),
            pl.BlockSpec((16, _H), lambda i: (0, 0)),
            pl.BlockSpec((1, _H), lambda i: (0, 0)),
            pl.BlockSpec((_H, _H), lambda i: (0, 0)),
            pl.BlockSpec((1, _H), lambda i: (0, 0)),
        ],
        out_specs=[pl.BlockSpec((_BE, _HH), lambda i: (i, 0))] * 2,
        out_shape=[out, out],
    )(asrc, bdst, ea, we_ea, b1, w2, b2)


def _node_update(h_ref, hw_ref, m0_ref, m1_ref, w1t_ref, w1b_ref, b1_ref,
                 w2_ref, b2_ref):
    t = (hw_ref[...] + _dotf(m0_ref[...], w1t_ref[...])
         + _dotf(m1_ref[...], w1b_ref[...]) + b1_ref[...])
    t = jnp.maximum(t, 0.0)
    return h_ref[...] + _dotf(t, w2_ref[...]) + b2_ref[...]


def _node_body(h_ref, hw_ref, m0_ref, m1_ref, w1t_ref, w1b_ref, b1_ref,
               w2_ref, b2_ref, o_ref):
    o_ref[...] = _node_update(h_ref, hw_ref, m0_ref, m1_ref, w1t_ref,
                              w1b_ref, b1_ref, w2_ref, b2_ref)


def _node_p_body(h_ref, hw_ref, m0_ref, m1_ref, w1t_ref, w1b_ref, b1_ref,
                 w2_ref, b2_ref, wc_ref, o_ref, a_ref, b_ref, c_ref):
    hn = _node_update(h_ref, hw_ref, m0_ref, m1_ref, w1t_ref, w1b_ref,
                      b1_ref, w2_ref, b2_ref)
    o_ref[...] = hn
    y = _dotf(hn, wc_ref[...])
    a_ref[...] = _pack16(y, 0)
    b_ref[...] = _pack16(y, _H)
    c_ref[...] = y[:, 2 * _H:]


def _tc_node(h, hw, m0, m1, w1t, w1b, b1, w2, b2, wcat_next=None):
    ins = [h, hw, m0, m1, w1t, w1b, b1, w2, b2]
    in_specs = [
        pl.BlockSpec((_BNP, _H), lambda i: (i, 0)),
        pl.BlockSpec((_BNP, _H), lambda i: (i, 0)),
        pl.BlockSpec((_BNP, _HH), lambda i: (i, 0)),
        pl.BlockSpec((_BNP, _HH), lambda i: (i, 0)),
        pl.BlockSpec((_HH, _H), lambda i: (0, 0)),
        pl.BlockSpec((_HH, _H), lambda i: (0, 0)),
        pl.BlockSpec((1, _H), lambda i: (0, 0)),
        pl.BlockSpec((_H, _H), lambda i: (0, 0)),
        pl.BlockSpec((1, _H), lambda i: (0, 0)),
    ]
    outf = jax.ShapeDtypeStruct((_N, _H), _F32)
    if wcat_next is None:
        return pl.pallas_call(
            _node_body,
            grid=(_NBP,),
            in_specs=in_specs,
            out_specs=pl.BlockSpec((_BNP, _H), lambda i: (i, 0)),
            out_shape=outf,
        )(*ins)
    outb = jax.ShapeDtypeStruct((_N, _HH), jnp.uint32)
    return pl.pallas_call(
        _node_p_body,
        grid=(_NBP,),
        in_specs=in_specs + [pl.BlockSpec((_H, 3 * _H), lambda i: (0, 0))],
        out_specs=[
            pl.BlockSpec((_BNP, _H), lambda i: (i, 0)),
            pl.BlockSpec((_BNP, _HH), lambda i: (i, 0)),
            pl.BlockSpec((_BNP, _HH), lambda i: (i, 0)),
            pl.BlockSpec((_BNP, _H), lambda i: (i, 0)),
        ],
        out_shape=[outf, outb, outb, outf],
    )(*ins, wcat_next)


def _final_body(h_ref, u_ref, wph_ref, wpu_ref, b1_ref, w2_ref, b2_ref,
                w3_ref, b3_ref, o_ref, acc_ref):
    i = pl.program_id(0)

    @pl.when(i == 0)
    def _():
        acc_ref[...] = jnp.zeros_like(acc_ref)

    acc_ref[...] += jnp.sum(h_ref[...], axis=0, keepdims=True)

    @pl.when(i == pl.num_programs(0) - 1)
    def _():
        gr = acc_ref[...] * (1.0 / _N)
        z = jnp.maximum(_dotf(gr, wph_ref[...]) + _dotf(u_ref[...], wpu_ref[...])
                        + b1_ref[...], 0.0)
        z = jnp.maximum(_dotf(z, w2_ref[...]) + b2_ref[...], 0.0)
        o_ref[...] = _dotf(z, w3_ref[...]) + b3_ref[...]


def _tc_final(h, u, wph, wpu, b1, w2, b2, w3, b3):
    return pl.pallas_call(
        _final_body,
        grid=(_NB,),
        in_specs=[
            pl.BlockSpec((_BN, _H), lambda i: (i, 0)),
            pl.BlockSpec((1, 4), lambda i: (0, 0)),
            pl.BlockSpec((_H, _H), lambda i: (0, 0)),
            pl.BlockSpec((4, _H), lambda i: (0, 0)),
            pl.BlockSpec((1, _H), lambda i: (0, 0)),
            pl.BlockSpec((_H, _HH), lambda i: (0, 0)),
            pl.BlockSpec((1, _HH), lambda i: (0, 0)),
            pl.BlockSpec((_HH, 3), lambda i: (0, 0)),
            pl.BlockSpec((1, 3), lambda i: (0, 0)),
        ],
        out_specs=pl.BlockSpec((1, 3), lambda i: (0, 0)),
        out_shape=jax.ShapeDtypeStruct((1, 3), _F32),
        scratch_shapes=[pltpu.VMEM((1, _H), _F32)],
    )(h, u, wph, wpu, b1, w2, b2, w3, b3)


def _pipeline(n_full, g_desc, w_desc, r0, sg0, sw0, r1, sg1, sw1):
    g_desc(0, r0, sg0).start()

    def body(j, carry):
        i0 = 2 * j
        i1 = i0 + 1

        @pl.when(j > 0)
        def _():
            w_desc(i0 - 1, r1, sw1).wait()

        g_desc(i0, r0, sg0).wait()
        g_desc(i1, r1, sg1).start()
        w_desc(i0, r0, sw0).start()
        g_desc(i1, r1, sg1).wait()
        w_desc(i0, r0, sw0).wait()

        @pl.when(i0 + 2 < n_full)
        def _():
            g_desc(i0 + 2, r0, sg0).start()

        w_desc(i1, r1, sw1).start()
        return carry

    lax.fori_loop(0, n_full // 2, body, 0)
    w_desc(n_full - 1, r1, sw1).wait()


def _gather_body(a_hbm, b_hbm, src_hbm, dst_hbm, ao_hbm, bo_hbm,
                 idx_v, rows0, rows1, sg0, sg1, sw0, sw1):
    c = lax.axis_index("c")
    s = lax.axis_index("s")
    base = s * _EPT

    def run(tab, idxs, out):
        pltpu.sync_copy(idxs.at[pl.ds(pl.multiple_of(base, 8), _EPT)], idx_v)

        def g_desc(i, buf, sem, n=_CH):
            io = pl.multiple_of(i * _CH, 8)
            return pltpu.make_async_copy(
                tab.at[idx_v.at[pl.ds(io, n)]], buf, sem)

        def w_desc(i, buf, sem, n=_CH):
            off = pl.multiple_of(base + i * _CH, 8)
            return pltpu.make_async_copy(buf, out.at[pl.ds(off, n)], sem)

        _pipeline(_NCHF, g_desc, w_desc, rows0, sg0, sw0, rows1, sg1, sw1)

        tl = rows0.at[pl.ds(0, _CHT)]
        g_desc(_NCHF, tl, sg0, _CHT).start()
        g_desc(_NCHF, tl, sg0, _CHT).wait()
        w_desc(_NCHF, tl, sw0, _CHT).start()
        w_desc(_NCHF, tl, sw0, _CHT).wait()

    @pl.when(c == 0)
    def _():
        run(a_hbm, src_hbm, ao_hbm)

    @pl.when(c == 1)
    def _():
        run(b_hbm, dst_hbm, bo_hbm)


def _sc_gather(a, b, src, dst):
    mesh = plsc.VectorSubcoreMesh(core_axis_name="c", subcore_axis_name="s")
    out = jax.ShapeDtypeStruct((_E, _HH), jnp.uint32)
    f = functools.partial(
        pl.kernel,
        mesh=mesh,
        out_type=(out, out),
        scratch_types=[
            pltpu.VMEM((_EPT,), jnp.int32),
            pltpu.VMEM((_CH, _HH), jnp.uint32),
            pltpu.VMEM((_CH, _HH), jnp.uint32),
            pltpu.SemaphoreType.DMA,
            pltpu.SemaphoreType.DMA,
            pltpu.SemaphoreType.DMA,
            pltpu.SemaphoreType.DMA,
        ],
    )(_gather_body)
    return f(a, b, src, dst)


def _scatter_body(m0_hbm, m1_hbm, dst_hbm, zero_hbm, o0_hbm, o1_hbm,
                  idx0, idx1, idxt, buf0, buf1, buft, sl0, sl1, acc_sh):
    c = lax.axis_index("c")
    s = lax.axis_index("s")
    r0 = pl.multiple_of(s * _RPT, 8)
    pltpu.sync_copy(zero_hbm, acc_sh.at[pl.ds(r0, _RPT)])
    plsc.subcore_barrier()

    base = s * _EPT

    def run(m_hbm):
        def descs(i, bufs, sem, n=_CH):
            idxb, buf = bufs
            off = pl.multiple_of(base + i * _CH, 8)
            return (pltpu.make_async_copy(dst_hbm.at[pl.ds(off, n)],
                                          idxb, sem),
                    pltpu.make_async_copy(m_hbm.at[pl.ds(off, n)], buf, sem))

        class _Loads:
            def __init__(self, i, bufs, sem, n=_CH):
                self.d = descs(i, bufs, sem, n)

            def start(self):
                for d in self.d:
                    d.start()

            def wait(self):
                for d in self.d:
                    d.wait()

        class _Scat:
            def __init__(self, i, bufs, sem, n=_CH):
                self.bufs = bufs

            def start(self):
                idxb, buf = self.bufs
                pltpu.sync_copy(buf, acc_sh.at[idxb], add=True)

            def wait(self):
                pass

        _pipeline(_NCHF,
                  lambda i, bufs, sem, n=_CH: _Loads(i, bufs, sem, n),
                  lambda i, bufs, sem, n=_CH: _Scat(i, bufs, sem, n),
                  (idx0, buf0), sl0, sl0, (idx1, buf1), sl1, sl1)

        tb = (idxt, buft)
        _Loads(_NCHF, tb, sl0, _CHT).start()
        _Loads(_NCHF, tb, sl0, _CHT).wait()
        _Scat(_NCHF, tb, sl0, _CHT).start()

    @pl.when(c == 0)
    def _():
        run(m0_hbm)

    @pl.when(c == 1)
    def _():
        run(m1_hbm)

    plsc.subcore_barrier()

    @pl.when(jnp.logical_and(c == 0, s < _NT - 1))
    def _():
        pltpu.sync_copy(acc_sh.at[pl.ds(r0, _RPT)], o0_hbm.at[pl.ds(r0, _RPT)])

    @pl.when(jnp.logical_and(c == 1, s < _NT - 1))
    def _():
        pltpu.sync_copy(acc_sh.at[pl.ds(r0, _RPT)], o1_hbm.at[pl.ds(r0, _RPT)])

    @pl.when(jnp.logical_and(c == 0, s == _NT - 1))
    def _():
        pltpu.sync_copy(acc_sh.at[pl.ds(r0, _RPT_LAST)],
                        o0_hbm.at[pl.ds(r0, _RPT_LAST)])

    @pl.when(jnp.logical_and(c == 1, s == _NT - 1))
    def _():
        pltpu.sync_copy(acc_sh.at[pl.ds(r0, _RPT_LAST)],
                        o1_hbm.at[pl.ds(r0, _RPT_LAST)])


def _sc_scatter(m0, m1, dst, zeros_half):
    mesh = plsc.VectorSubcoreMesh(core_axis_name="c", subcore_axis_name="s")
    out = jax.ShapeDtypeStruct((_N, _HH), _F32)
    f = functools.partial(
        pl.kernel,
        mesh=mesh,
        out_type=(out, out),
        scratch_types=[
            pltpu.VMEM((_CH,), jnp.int32),
            pltpu.VMEM((_CH,), jnp.int32),
            pltpu.VMEM((_CHT,), jnp.int32),
            pltpu.VMEM((_CH, _HH), _F32),
            pltpu.VMEM((_CH, _HH), _F32),
            pltpu.VMEM((_CHT, _HH), _F32),
            pltpu.SemaphoreType.DMA,
            pltpu.SemaphoreType.DMA,
            pltpu.VMEM_SHARED((_ACC_N, _HH), _F32),
        ],
    )(_scatter_body)
    return f(m0, m1, dst, zeros_half)


def kernel(x, edge_index, edge_attr, u, batch, W_embed, b_embed, Wrel, brel,
           Wroot, We1, be1, We2, be2, Wn1, bn1, Wn2, bn2, Wp1, bp1, Wp2, bp2,
           Wp3, bp3):
    src = edge_index[0]
    dst = edge_index[1]

    x8 = jnp.pad(x, ((0, 0), (0, 1)))
    w8 = jnp.pad(W_embed, ((0, 1), (0, 0)))

    zeros_half = jnp.zeros((_RPT, _HH), _F32)
    ea16 = edge_attr.astype(jnp.bfloat16)
    wcats = [jnp.concatenate([We1[i][:_H], We1[i][_H:2 * _H], Wn1[i][:_H]],
                             axis=1) for i in range(4)]

    h, a, b, hw = _tc_embed_p(x8, w8, b_embed.reshape(1, _H), wcats[0])

    for i in range(4):
        asrc, bdst = _sc_gather(a, b, src, dst)
        m0, m1 = _tc_edge(asrc, bdst, ea16,
                          We1[i][2 * _H:].astype(jnp.bfloat16),
                          be1[i].reshape(1, _H), We2[i].astype(jnp.bfloat16),
                          be2[i].reshape(1, _H))
        g0, g1 = _sc_scatter(m0, m1, dst, zeros_half)
        nxt = _tc_node(h, hw, g0, g1, Wn1[i][_H:_H + _HH], Wn1[i][_H + _HH:],
                       bn1[i].reshape(1, _H), Wn2[i], bn2[i].reshape(1, _H),
                       wcats[i + 1] if i < 3 else None)
        if i < 3:
            h, a, b, hw = nxt
        else:
            h = nxt

    return _tc_final(h, u, Wp1[:_H], Wp1[_H:], bp1.reshape(1, _H),
                     Wp2, bp2.reshape(1, _HH), Wp3, bp3.reshape(1, 3))


# speedup vs baseline: 1.4506x; 1.0513x over previous
import functools

import jax
import jax.numpy as jnp
from jax import lax
from jax.experimental import pallas as pl
from jax.experimental.pallas import tpu as pltpu
from jax.experimental.pallas import tpu_sc as plsc

_N = 10000
_E = 160000
_H = 256
_HH = 128

_NB = 10
_BN = _N // _NB
_NBP = 5
_BNP = _N // _NBP
_EB = 40
_BE = _E // _EB

_NT = 16
_CH = 128
_EPT = _E // _NT
_NCHF = _EPT // _CH
_CHT = _EPT - _NCHF * _CH
_ACC_N = 10240
_RPT = _ACC_N // _NT
_RPT_LAST = _N - (_NT - 1) * _RPT

_F32 = jnp.float32


def _dotf(a, b):
    return jnp.dot(a, b, preferred_element_type=_F32)


def _embed_body(x_ref, w_ref, b_ref, wc_ref, h_ref, a_ref, b2_ref, c_ref):
    h = _dotf(x_ref[...], w_ref[...]) + b_ref[...]
    h_ref[...] = h
    y = _dotf(h, wc_ref[...])
    a_ref[...] = _pack16(y, 0)
    b2_ref[...] = _pack16(y, _H)
    c_ref[...] = y[:, 2 * _H:]


def _tc_embed_p(x8, w8, b, wcat):
    outb = jax.ShapeDtypeStruct((_N, _HH), jnp.uint32)
    outf = jax.ShapeDtypeStruct((_N, _H), _F32)
    return pl.pallas_call(
        _embed_body,
        grid=(_NBP,),
        in_specs=[
            pl.BlockSpec((_BNP, 8), lambda i: (i, 0)),
            pl.BlockSpec((8, _H), lambda i: (0, 0)),
            pl.BlockSpec((1, _H), lambda i: (0, 0)),
            pl.BlockSpec((_H, 3 * _H), lambda i: (0, 0)),
        ],
        out_specs=[
            pl.BlockSpec((_BNP, _H), lambda i: (i, 0)),
            pl.BlockSpec((_BNP, _HH), lambda i: (i, 0)),
            pl.BlockSpec((_BNP, _HH), lambda i: (i, 0)),
            pl.BlockSpec((_BNP, _H), lambda i: (i, 0)),
        ],
        out_shape=[outf, outb, outb, outf],
    )(x8, w8, b, wcat)


def _pack16(y, lo0):
    lo = jax.lax.bitcast_convert_type(y[:, lo0:lo0 + _HH], jnp.uint32)
    hi = jax.lax.bitcast_convert_type(y[:, lo0 + _HH:lo0 + 2 * _HH], jnp.uint32)
    rnd = jnp.uint32(0x8000)
    return ((hi + rnd) & jnp.uint32(0xFFFF0000)) | ((lo + rnd) >> 16)


def _unpack16(x):
    lo = jax.lax.bitcast_convert_type(x << 16, _F32)
    hi = jax.lax.bitcast_convert_type(x & jnp.uint32(0xFFFF0000), _F32)
    return lo, hi


def _p_body(h_ref, w_ref, a_ref, b_ref, c_ref):
    y = _dotf(h_ref[...], w_ref[...])
    a_ref[...] = _pack16(y, 0)
    b_ref[...] = _pack16(y, _H)
    c_ref[...] = y[:, 2 * _H:]


def _tc_p(h, wcat):
    outb = jax.ShapeDtypeStruct((_N, _HH), jnp.uint32)
    outf = jax.ShapeDtypeStruct((_N, _H), _F32)
    return pl.pallas_call(
        _p_body,
        grid=(_NBP,),
        in_specs=[
            pl.BlockSpec((_BNP, _H), lambda i: (i, 0)),
            pl.BlockSpec((_H, 3 * _H), lambda i: (0, 0)),
        ],
        out_specs=[
            pl.BlockSpec((_BNP, _HH), lambda i: (i, 0)),
            pl.BlockSpec((_BNP, _HH), lambda i: (i, 0)),
            pl.BlockSpec((_BNP, _H), lambda i: (i, 0)),
        ],
        out_shape=[outb, outb, outf],
    )(h, wcat)


def _edge_body(a_ref, b_ref, ea_ref, we_ref, b1_ref, w2_ref, b2_ref,
               m0_ref, m1_ref):
    alo, ahi = _unpack16(a_ref[...])
    blo, bhi = _unpack16(b_ref[...])
    cc = _dotf(ea_ref[...], we_ref[...]) + b1_ref[...]
    tlo = jnp.maximum(alo + blo + cc[:, :_HH], 0.0)
    thi = jnp.maximum(ahi + bhi + cc[:, _HH:], 0.0)
    t = jnp.concatenate([tlo, thi], axis=1).astype(jnp.bfloat16)
    m = _dotf(t, w2_ref[...]) + b2_ref[...]
    m0_ref[...] = m[:, :_HH]
    m1_ref[...] = m[:, _HH:]


def _tc_edge(asrc, bdst, ea, we_ea, b1, w2, b2):
    out = jax.ShapeDtypeStruct((_E, _HH), _F32)
    return pl.pallas_call(
        _edge_body,
        grid=(_EB,),
        in_specs=[
            pl.BlockSpec((_BE, _HH), lambda i: (i, 0)),
            pl.BlockSpec((_BE, _HH), lambda i: (i, 0)),
            pl.BlockSpec((_BE, 16), lambda i: (i, 0)),
            pl.BlockSpec((16, _H), lambda i: (0, 0)),
            pl.BlockSpec((1, _H), lambda i: (0, 0)),
            pl.BlockSpec((_H, _H), lambda i: (0, 0)),
            pl.BlockSpec((1, _H), lambda i: (0, 0)),
        ],
        out_specs=[pl.BlockSpec((_BE, _HH), lambda i: (i, 0))] * 2,
        out_shape=[out, out],
    )(asrc, bdst, ea, we_ea, b1, w2, b2)


def _node_update(h_ref, hw_ref, m0_ref, m1_ref, w1t_ref, w1b_ref, b1_ref,
                 w2_ref, b2_ref):
    t = (hw_ref[...] + _dotf(m0_ref[...], w1t_ref[...])
         + _dotf(m1_ref[...], w1b_ref[...]) + b1_ref[...])
    t = jnp.maximum(t, 0.0)
    return h_ref[...] + _dotf(t, w2_ref[...]) + b2_ref[...]


def _node_body(h_ref, hw_ref, m0_ref, m1_ref, w1t_ref, w1b_ref, b1_ref,
               w2_ref, b2_ref, o_ref):
    o_ref[...] = _node_update(h_ref, hw_ref, m0_ref, m1_ref, w1t_ref,
                              w1b_ref, b1_ref, w2_ref, b2_ref)


def _node_p_body(h_ref, hw_ref, m0_ref, m1_ref, w1t_ref, w1b_ref, b1_ref,
                 w2_ref, b2_ref, wc_ref, o_ref, a_ref, b_ref, c_ref):
    hn = _node_update(h_ref, hw_ref, m0_ref, m1_ref, w1t_ref, w1b_ref,
                      b1_ref, w2_ref, b2_ref)
    o_ref[...] = hn
    y = _dotf(hn, wc_ref[...])
    a_ref[...] = _pack16(y, 0)
    b_ref[...] = _pack16(y, _H)
    c_ref[...] = y[:, 2 * _H:]


def _tc_node(h, hw, m0, m1, w1t, w1b, b1, w2, b2, wcat_next=None):
    ins = [h, hw, m0, m1, w1t, w1b, b1, w2, b2]
    in_specs = [
        pl.BlockSpec((_BNP, _H), lambda i: (i, 0)),
        pl.BlockSpec((_BNP, _H), lambda i: (i, 0)),
        pl.BlockSpec((_BNP, _HH), lambda i: (i, 0)),
        pl.BlockSpec((_BNP, _HH), lambda i: (i, 0)),
        pl.BlockSpec((_HH, _H), lambda i: (0, 0)),
        pl.BlockSpec((_HH, _H), lambda i: (0, 0)),
        pl.BlockSpec((1, _H), lambda i: (0, 0)),
        pl.BlockSpec((_H, _H), lambda i: (0, 0)),
        pl.BlockSpec((1, _H), lambda i: (0, 0)),
    ]
    outf = jax.ShapeDtypeStruct((_N, _H), _F32)
    if wcat_next is None:
        return pl.pallas_call(
            _node_body,
            grid=(_NBP,),
            in_specs=in_specs,
            out_specs=pl.BlockSpec((_BNP, _H), lambda i: (i, 0)),
            out_shape=outf,
        )(*ins)
    outb = jax.ShapeDtypeStruct((_N, _HH), jnp.uint32)
    return pl.pallas_call(
        _node_p_body,
        grid=(_NBP,),
        in_specs=in_specs + [pl.BlockSpec((_H, 3 * _H), lambda i: (0, 0))],
        out_specs=[
            pl.BlockSpec((_BNP, _H), lambda i: (i, 0)),
            pl.BlockSpec((_BNP, _HH), lambda i: (i, 0)),
            pl.BlockSpec((_BNP, _HH), lambda i: (i, 0)),
            pl.BlockSpec((_BNP, _H), lambda i: (i, 0)),
        ],
        out_shape=[outf, outb, outb, outf],
    )(*ins, wcat_next)


def _final_body(h_ref, u_ref, wph_ref, wpu_ref, b1_ref, w2_ref, b2_ref,
                w3_ref, b3_ref, o_ref, acc_ref):
    i = pl.program_id(0)

    @pl.when(i == 0)
    def _():
        acc_ref[...] = jnp.zeros_like(acc_ref)

    acc_ref[...] += jnp.sum(h_ref[...], axis=0, keepdims=True)

    @pl.when(i == pl.num_programs(0) - 1)
    def _():
        gr = acc_ref[...] * (1.0 / _N)
        z = jnp.maximum(_dotf(gr, wph_ref[...]) + _dotf(u_ref[...], wpu_ref[...])
                        + b1_ref[...], 0.0)
        z = jnp.maximum(_dotf(z, w2_ref[...]) + b2_ref[...], 0.0)
        o_ref[...] = _dotf(z, w3_ref[...]) + b3_ref[...]


def _tc_final(h, u, wph, wpu, b1, w2, b2, w3, b3):
    return pl.pallas_call(
        _final_body,
        grid=(_NB,),
        in_specs=[
            pl.BlockSpec((_BN, _H), lambda i: (i, 0)),
            pl.BlockSpec((1, 4), lambda i: (0, 0)),
            pl.BlockSpec((_H, _H), lambda i: (0, 0)),
            pl.BlockSpec((4, _H), lambda i: (0, 0)),
            pl.BlockSpec((1, _H), lambda i: (0, 0)),
            pl.BlockSpec((_H, _HH), lambda i: (0, 0)),
            pl.BlockSpec((1, _HH), lambda i: (0, 0)),
            pl.BlockSpec((_HH, 3), lambda i: (0, 0)),
            pl.BlockSpec((1, 3), lambda i: (0, 0)),
        ],
        out_specs=pl.BlockSpec((1, 3), lambda i: (0, 0)),
        out_shape=jax.ShapeDtypeStruct((1, 3), _F32),
        scratch_shapes=[pltpu.VMEM((1, _H), _F32)],
    )(h, u, wph, wpu, b1, w2, b2, w3, b3)


def _pipeline(n_full, g_desc, w_desc, r0, sg0, sw0, r1, sg1, sw1):
    g_desc(0, r0, sg0).start()

    def body(j, carry):
        i0 = 2 * j
        i1 = i0 + 1

        @pl.when(j > 0)
        def _():
            w_desc(i0 - 1, r1, sw1).wait()

        g_desc(i0, r0, sg0).wait()
        g_desc(i1, r1, sg1).start()
        w_desc(i0, r0, sw0).start()
        g_desc(i1, r1, sg1).wait()
        w_desc(i0, r0, sw0).wait()

        @pl.when(i0 + 2 < n_full)
        def _():
            g_desc(i0 + 2, r0, sg0).start()

        w_desc(i1, r1, sw1).start()
        return carry

    lax.fori_loop(0, n_full // 2, body, 0)
    w_desc(n_full - 1, r1, sw1).wait()


def _gather_body(a_hbm, b_hbm, src_hbm, dst_hbm, ao_hbm, bo_hbm,
                 idx_v, rows0, rows1, sg0, sg1, sw0, sw1):
    c = lax.axis_index("c")
    s = lax.axis_index("s")
    base = s * _EPT

    def run(tab, idxs, out):
        pltpu.sync_copy(idxs.at[pl.ds(pl.multiple_of(base, 8), _EPT)], idx_v)

        def g_desc(i, buf, sem, n=_CH):
            io = pl.multiple_of(i * _CH, 8)
            return pltpu.make_async_copy(
                tab.at[idx_v.at[pl.ds(io, n)]], buf, sem)

        def w_desc(i, buf, sem, n=_CH):
            off = pl.multiple_of(base + i * _CH, 8)
            return pltpu.make_async_copy(buf, out.at[pl.ds(off, n)], sem)

        _pipeline(_NCHF, g_desc, w_desc, rows0, sg0, sw0, rows1, sg1, sw1)

        tl = rows0.at[pl.ds(0, _CHT)]
        g_desc(_NCHF, tl, sg0, _CHT).start()
        g_desc(_NCHF, tl, sg0, _CHT).wait()
        w_desc(_NCHF, tl, sw0, _CHT).start()
        w_desc(_NCHF, tl, sw0, _CHT).wait()

    @pl.when(c == 0)
    def _():
        run(a_hbm, src_hbm, ao_hbm)

    @pl.when(c == 1)
    def _():
        run(b_hbm, dst_hbm, bo_hbm)


def _sc_gather(a, b, src, dst):
    mesh = plsc.VectorSubcoreMesh(core_axis_name="c", subcore_axis_name="s")
    out = jax.ShapeDtypeStruct((_E, _HH), jnp.uint32)
    f = functools.partial(
        pl.kernel,
        mesh=mesh,
        out_type=(out, out),
        scratch_types=[
            pltpu.VMEM((_EPT,), jnp.int32),
            pltpu.VMEM((_CH, _HH), jnp.uint32),
            pltpu.VMEM((_CH, _HH), jnp.uint32),
            pltpu.SemaphoreType.DMA,
            pltpu.SemaphoreType.DMA,
            pltpu.SemaphoreType.DMA,
            pltpu.SemaphoreType.DMA,
        ],
    )(_gather_body)
    return f(a, b, src, dst)


def _scatter_body(m0_hbm, m1_hbm, dst_hbm, zero_hbm, o0_hbm, o1_hbm,
                  idx0, idx1, idxt, buf0, buf1, buft, sl0, sl1, acc_sh):
    c = lax.axis_index("c")
    s = lax.axis_index("s")
    r0 = pl.multiple_of(s * _RPT, 8)
    pltpu.sync_copy(zero_hbm, acc_sh.at[pl.ds(r0, _RPT)])
    plsc.subcore_barrier()

    base = s * _EPT

    def run(m_hbm):
        def descs(i, bufs, sem, n=_CH):
            idxb, buf = bufs
            off = pl.multiple_of(base + i * _CH, 8)
            return (pltpu.make_async_copy(dst_hbm.at[pl.ds(off, n)],
                                          idxb, sem),
                    pltpu.make_async_copy(m_hbm.at[pl.ds(off, n)], buf, sem))

        class _Loads:
            def __init__(self, i, bufs, sem, n=_CH):
                self.d = descs(i, bufs, sem, n)

            def start(self):
                for d in self.d:
                    d.start()

            def wait(self):
                for d in self.d:
                    d.wait()

        class _Scat:
            def __init__(self, i, bufs, sem, n=_CH):
                self.bufs = bufs

            def start(self):
                idxb, buf = self.bufs
                pltpu.sync_copy(buf, acc_sh.at[idxb], add=True)

            def wait(self):
                pass

        _pipeline(_NCHF,
                  lambda i, bufs, sem, n=_CH: _Loads(i, bufs, sem, n),
                  lambda i, bufs, sem, n=_CH: _Scat(i, bufs, sem, n),
                  (idx0, buf0), sl0, sl0, (idx1, buf1), sl1, sl1)

        tb = (idxt, buft)
        _Loads(_NCHF, tb, sl0, _CHT).start()
        _Loads(_NCHF, tb, sl0, _CHT).wait()
        _Scat(_NCHF, tb, sl0, _CHT).start()

    @pl.when(c == 0)
    def _():
        run(m0_hbm)

    @pl.when(c == 1)
    def _():
        run(m1_hbm)

    plsc.subcore_barrier()

    @pl.when(jnp.logical_and(c == 0, s < _NT - 1))
    def _():
        pltpu.sync_copy(acc_sh.at[pl.ds(r0, _RPT)], o0_hbm.at[pl.ds(r0, _RPT)])

    @pl.when(jnp.logical_and(c == 1, s < _NT - 1))
    def _():
        pltpu.sync_copy(acc_sh.at[pl.ds(r0, _RPT)], o1_hbm.at[pl.ds(r0, _RPT)])

    @pl.when(jnp.logical_and(c == 0, s == _NT - 1))
    def _():
        pltpu.sync_copy(acc_sh.at[pl.ds(r0, _RPT_LAST)],
                        o0_hbm.at[pl.ds(r0, _RPT_LAST)])

    @pl.when(jnp.logical_and(c == 1, s == _NT - 1))
    def _():
        pltpu.sync_copy(acc_sh.at[pl.ds(r0, _RPT_LAST)],
                        o1_hbm.at[pl.ds(r0, _RPT_LAST)])


def _sc_scatter(m0, m1, dst, zeros_half):
    mesh = plsc.VectorSubcoreMesh(core_axis_name="c", subcore_axis_name="s")
    out = jax.ShapeDtypeStruct((_N, _HH), _F32)
    f = functools.partial(
        pl.kernel,
        mesh=mesh,
        out_type=(out, out),
        scratch_types=[
            pltpu.VMEM((_CH,), jnp.int32),
            pltpu.VMEM((_CH,), jnp.int32),
            pltpu.VMEM((_CHT,), jnp.int32),
            pltpu.VMEM((_CH, _HH), _F32),
            pltpu.VMEM((_CH, _HH), _F32),
            pltpu.VMEM((_CHT, _HH), _F32),
            pltpu.SemaphoreType.DMA,
            pltpu.SemaphoreType.DMA,
            pltpu.VMEM_SHARED((_ACC_N, _HH), _F32),
        ],
    )(_scatter_body)
    return f(m0, m1, dst, zeros_half)


def kernel(x, edge_index, edge_attr, u, batch, W_embed, b_embed, Wrel, brel,
           Wroot, We1, be1, We2, be2, Wn1, bn1, Wn2, bn2, Wp1, bp1, Wp2, bp2,
           Wp3, bp3):
    src = edge_index[0]
    dst = edge_index[1]

    x8 = jnp.pad(x, ((0, 0), (0, 1)))
    w8 = jnp.pad(W_embed, ((0, 1), (0, 0)))

    zeros_half = jnp.zeros((_RPT, _HH), _F32)
    ea16 = edge_attr.astype(jnp.bfloat16)
    wcats = [jnp.concatenate([We1[i][:_H], We1[i][_H:2 * _H], Wn1[i][:_H]],
                             axis=1) for i in range(4)]

    h, a, b, hw = _tc_embed_p(x8, w8, b_embed.reshape(1, _H), wcats[0])

    for i in range(4):
        asrc, bdst = _sc_gather(a, b, src, dst)
        m0, m1 = _tc_edge(asrc, bdst, ea16,
                          We1[i][2 * _H:].astype(jnp.bfloat16),
                          be1[i].reshape(1, _H), We2[i].astype(jnp.bfloat16),
                          be2[i].reshape(1, _H))
        g0, g1 = _sc_scatter(m0, m1, dst, zeros_half)
        nxt = _tc_node(h, hw, g0, g1, Wn1[i][_H:_H + _HH], Wn1[i][_H + _HH:],
                       bn1[i].reshape(1, _H), Wn2[i], bn2[i].reshape(1, _H),
                       wcats[i + 1] if i < 3 else None)
        if i < 3:
            h, a, b, hw = nxt
        else:
            h = nxt

    return _tc_final(h, u, Wp1[:_H], Wp1[_H:], bp1.reshape(1, _H),
                     Wp2, bp2.reshape(1, _HH), Wp3, bp3.reshape(1, 3))


# speedup vs baseline: 1.4568x; 1.0043x over previous
import functools

import jax
import jax.numpy as jnp
from jax import lax
from jax.experimental import pallas as pl
from jax.experimental.pallas import tpu as pltpu
from jax.experimental.pallas import tpu_sc as plsc

_N = 10000
_E = 160000
_H = 256
_HH = 128

_NB = 10
_BN = _N // _NB
_NBP = 5
_BNP = _N // _NBP
_EB = 20
_BE = _E // _EB

_NT = 16
_CH = 128
_EPT = _E // _NT
_NCHF = _EPT // _CH
_CHT = _EPT - _NCHF * _CH
_ACC_N = 10240
_RPT = _ACC_N // _NT
_RPT_LAST = _N - (_NT - 1) * _RPT

_F32 = jnp.float32


def _dotf(a, b):
    return jnp.dot(a, b, preferred_element_type=_F32)


def _embed_body(x_ref, w_ref, b_ref, wc_ref, h_ref, a_ref, b2_ref, c_ref):
    h = _dotf(x_ref[...], w_ref[...]) + b_ref[...]
    h_ref[...] = h
    y = _dotf(h, wc_ref[...])
    a_ref[...] = _pack16(y, 0)
    b2_ref[...] = _pack16(y, _H)
    c_ref[...] = y[:, 2 * _H:]


def _tc_embed_p(x8, w8, b, wcat):
    outb = jax.ShapeDtypeStruct((_N, _HH), jnp.uint32)
    outf = jax.ShapeDtypeStruct((_N, _H), _F32)
    return pl.pallas_call(
        _embed_body,
        grid=(_NBP,),
        in_specs=[
            pl.BlockSpec((_BNP, 8), lambda i: (i, 0)),
            pl.BlockSpec((8, _H), lambda i: (0, 0)),
            pl.BlockSpec((1, _H), lambda i: (0, 0)),
            pl.BlockSpec((_H, 3 * _H), lambda i: (0, 0)),
        ],
        out_specs=[
            pl.BlockSpec((_BNP, _H), lambda i: (i, 0)),
            pl.BlockSpec((_BNP, _HH), lambda i: (i, 0)),
            pl.BlockSpec((_BNP, _HH), lambda i: (i, 0)),
            pl.BlockSpec((_BNP, _H), lambda i: (i, 0)),
        ],
        out_shape=[outf, outb, outb, outf],
    )(x8, w8, b, wcat)


def _pack16(y, lo0):
    lo = jax.lax.bitcast_convert_type(y[:, lo0:lo0 + _HH], jnp.uint32)
    hi = jax.lax.bitcast_convert_type(y[:, lo0 + _HH:lo0 + 2 * _HH], jnp.uint32)
    rnd = jnp.uint32(0x8000)
    return ((hi + rnd) & jnp.uint32(0xFFFF0000)) | ((lo + rnd) >> 16)


def _unpack16(x):
    lo = jax.lax.bitcast_convert_type(x << 16, _F32)
    hi = jax.lax.bitcast_convert_type(x & jnp.uint32(0xFFFF0000), _F32)
    return lo, hi


def _p_body(h_ref, w_ref, a_ref, b_ref, c_ref):
    y = _dotf(h_ref[...], w_ref[...])
    a_ref[...] = _pack16(y, 0)
    b_ref[...] = _pack16(y, _H)
    c_ref[...] = y[:, 2 * _H:]


def _tc_p(h, wcat):
    outb = jax.ShapeDtypeStruct((_N, _HH), jnp.uint32)
    outf = jax.ShapeDtypeStruct((_N, _H), _F32)
    return pl.pallas_call(
        _p_body,
        grid=(_NBP,),
        in_specs=[
            pl.BlockSpec((_BNP, _H), lambda i: (i, 0)),
            pl.BlockSpec((_H, 3 * _H), lambda i: (0, 0)),
        ],
        out_specs=[
            pl.BlockSpec((_BNP, _HH), lambda i: (i, 0)),
            pl.BlockSpec((_BNP, _HH), lambda i: (i, 0)),
            pl.BlockSpec((_BNP, _H), lambda i: (i, 0)),
        ],
        out_shape=[outb, outb, outf],
    )(h, wcat)


def _edge_body(a_ref, b_ref, ea_ref, we_ref, b1_ref, w2_ref, b2_ref,
               m0_ref, m1_ref):
    alo, ahi = _unpack16(a_ref[...])
    blo, bhi = _unpack16(b_ref[...])
    cc = _dotf(ea_ref[...], we_ref[...]) + b1_ref[...]
    tlo = jnp.maximum(alo + blo + cc[:, :_HH], 0.0)
    thi = jnp.maximum(ahi + bhi + cc[:, _HH:], 0.0)
    t = jnp.concatenate([tlo, thi], axis=1).astype(jnp.bfloat16)
    m = _dotf(t, w2_ref[...]) + b2_ref[...]
    m0_ref[...] = m[:, :_HH]
    m1_ref[...] = m[:, _HH:]


def _tc_edge(asrc, bdst, ea, we_ea, b1, w2, b2):
    out = jax.ShapeDtypeStruct((_E, _HH), _F32)
    return pl.pallas_call(
        _edge_body,
        grid=(_EB,),
        in_specs=[
            pl.BlockSpec((_BE, _HH), lambda i: (i, 0)),
            pl.BlockSpec((_BE, _HH), lambda i: (i, 0)),
            pl.BlockSpec((_BE, 16), lambda i: (i, 0)),
            pl.BlockSpec((16, _H), lambda i: (0, 0)),
            pl.BlockSpec((1, _H), lambda i: (0, 0)),
            pl.BlockSpec((_H, _H), lambda i: (0, 0)),
            pl.BlockSpec((1, _H), lambda i: (0, 0)),
        ],
        out_specs=[pl.BlockSpec((_BE, _HH), lambda i: (i, 0))] * 2,
        out_shape=[out, out],
    )(asrc, bdst, ea, we_ea, b1, w2, b2)


def _node_update(h_ref, hw_ref, m0_ref, m1_ref, w1t_ref, w1b_ref, b1_ref,
                 w2_ref, b2_ref):
    t = (hw_ref[...] + _dotf(m0_ref[...], w1t_ref[...])
         + _dotf(m1_ref[...], w1b_ref[...]) + b1_ref[...])
    t = jnp.maximum(t, 0.0)
    return h_ref[...] + _dotf(t, w2_ref[...]) + b2_ref[...]


def _node_body(h_ref, hw_ref, m0_ref, m1_ref, w1t_ref, w1b_ref, b1_ref,
               w2_ref, b2_ref, o_ref):
    o_ref[...] = _node_update(h_ref, hw_ref, m0_ref, m1_ref, w1t_ref,
                              w1b_ref, b1_ref, w2_ref, b2_ref)


def _node_p_body(h_ref, hw_ref, m0_ref, m1_ref, w1t_ref, w1b_ref, b1_ref,
                 w2_ref, b2_ref, wc_ref, o_ref, a_ref, b_ref, c_ref):
    hn = _node_update(h_ref, hw_ref, m0_ref, m1_ref, w1t_ref, w1b_ref,
                      b1_ref, w2_ref, b2_ref)
    o_ref[...] = hn
    y = _dotf(hn, wc_ref[...])
    a_ref[...] = _pack16(y, 0)
    b_ref[...] = _pack16(y, _H)
    c_ref[...] = y[:, 2 * _H:]


def _tc_node(h, hw, m0, m1, w1t, w1b, b1, w2, b2, wcat_next=None):
    ins = [h, hw, m0, m1, w1t, w1b, b1, w2, b2]
    in_specs = [
        pl.BlockSpec((_BNP, _H), lambda i: (i, 0)),
        pl.BlockSpec((_BNP, _H), lambda i: (i, 0)),
        pl.BlockSpec((_BNP, _HH), lambda i: (i, 0)),
        pl.BlockSpec((_BNP, _HH), lambda i: (i, 0)),
        pl.BlockSpec((_HH, _H), lambda i: (0, 0)),
        pl.BlockSpec((_HH, _H), lambda i: (0, 0)),
        pl.BlockSpec((1, _H), lambda i: (0, 0)),
        pl.BlockSpec((_H, _H), lambda i: (0, 0)),
        pl.BlockSpec((1, _H), lambda i: (0, 0)),
    ]
    outf = jax.ShapeDtypeStruct((_N, _H), _F32)
    if wcat_next is None:
        return pl.pallas_call(
            _node_body,
            grid=(_NBP,),
            in_specs=in_specs,
            out_specs=pl.BlockSpec((_BNP, _H), lambda i: (i, 0)),
            out_shape=outf,
        )(*ins)
    outb = jax.ShapeDtypeStruct((_N, _HH), jnp.uint32)
    return pl.pallas_call(
        _node_p_body,
        grid=(_NBP,),
        in_specs=in_specs + [pl.BlockSpec((_H, 3 * _H), lambda i: (0, 0))],
        out_specs=[
            pl.BlockSpec((_BNP, _H), lambda i: (i, 0)),
            pl.BlockSpec((_BNP, _HH), lambda i: (i, 0)),
            pl.BlockSpec((_BNP, _HH), lambda i: (i, 0)),
            pl.BlockSpec((_BNP, _H), lambda i: (i, 0)),
        ],
        out_shape=[outf, outb, outb, outf],
    )(*ins, wcat_next)


def _final_body(h_ref, u_ref, wph_ref, wpu_ref, b1_ref, w2_ref, b2_ref,
                w3_ref, b3_ref, o_ref, acc_ref):
    i = pl.program_id(0)

    @pl.when(i == 0)
    def _():
        acc_ref[...] = jnp.zeros_like(acc_ref)

    acc_ref[...] += jnp.sum(h_ref[...], axis=0, keepdims=True)

    @pl.when(i == pl.num_programs(0) - 1)
    def _():
        gr = acc_ref[...] * (1.0 / _N)
        z = jnp.maximum(_dotf(gr, wph_ref[...]) + _dotf(u_ref[...], wpu_ref[...])
                        + b1_ref[...], 0.0)
        z = jnp.maximum(_dotf(z, w2_ref[...]) + b2_ref[...], 0.0)
        o_ref[...] = _dotf(z, w3_ref[...]) + b3_ref[...]


def _tc_final(h, u, wph, wpu, b1, w2, b2, w3, b3):
    return pl.pallas_call(
        _final_body,
        grid=(_NB,),
        in_specs=[
            pl.BlockSpec((_BN, _H), lambda i: (i, 0)),
            pl.BlockSpec((1, 4), lambda i: (0, 0)),
            pl.BlockSpec((_H, _H), lambda i: (0, 0)),
            pl.BlockSpec((4, _H), lambda i: (0, 0)),
            pl.BlockSpec((1, _H), lambda i: (0, 0)),
            pl.BlockSpec((_H, _HH), lambda i: (0, 0)),
            pl.BlockSpec((1, _HH), lambda i: (0, 0)),
            pl.BlockSpec((_HH, 3), lambda i: (0, 0)),
            pl.BlockSpec((1, 3), lambda i: (0, 0)),
        ],
        out_specs=pl.BlockSpec((1, 3), lambda i: (0, 0)),
        out_shape=jax.ShapeDtypeStruct((1, 3), _F32),
        scratch_shapes=[pltpu.VMEM((1, _H), _F32)],
    )(h, u, wph, wpu, b1, w2, b2, w3, b3)


def _pipeline(n_full, g_desc, w_desc, r0, sg0, sw0, r1, sg1, sw1):
    g_desc(0, r0, sg0).start()

    def body(j, carry):
        i0 = 2 * j
        i1 = i0 + 1

        @pl.when(j > 0)
        def _():
            w_desc(i0 - 1, r1, sw1).wait()

        g_desc(i0, r0, sg0).wait()
        g_desc(i1, r1, sg1).start()
        w_desc(i0, r0, sw0).start()
        g_desc(i1, r1, sg1).wait()
        w_desc(i0, r0, sw0).wait()

        @pl.when(i0 + 2 < n_full)
        def _():
            g_desc(i0 + 2, r0, sg0).start()

        w_desc(i1, r1, sw1).start()
        return carry

    lax.fori_loop(0, n_full // 2, body, 0)
    w_desc(n_full - 1, r1, sw1).wait()


def _gather_body(a_hbm, b_hbm, src_hbm, dst_hbm, ao_hbm, bo_hbm,
                 idx_v, rows0, rows1, sg0, sg1, sw0, sw1):
    c = lax.axis_index("c")
    s = lax.axis_index("s")
    base = s * _EPT

    def run(tab, idxs, out):
        pltpu.sync_copy(idxs.at[pl.ds(pl.multiple_of(base, 8), _EPT)], idx_v)

        def g_desc(i, buf, sem, n=_CH):
            io = pl.multiple_of(i * _CH, 8)
            return pltpu.make_async_copy(
                tab.at[idx_v.at[pl.ds(io, n)]], buf, sem)

        def w_desc(i, buf, sem, n=_CH):
            off = pl.multiple_of(base + i * _CH, 8)
            return pltpu.make_async_copy(buf, out.at[pl.ds(off, n)], sem)

        _pipeline(_NCHF, g_desc, w_desc, rows0, sg0, sw0, rows1, sg1, sw1)

        tl = rows0.at[pl.ds(0, _CHT)]
        g_desc(_NCHF, tl, sg0, _CHT).start()
        g_desc(_NCHF, tl, sg0, _CHT).wait()
        w_desc(_NCHF, tl, sw0, _CHT).start()
        w_desc(_NCHF, tl, sw0, _CHT).wait()

    @pl.when(c == 0)
    def _():
        run(a_hbm, src_hbm, ao_hbm)

    @pl.when(c == 1)
    def _():
        run(b_hbm, dst_hbm, bo_hbm)


def _sc_gather(a, b, src, dst):
    mesh = plsc.VectorSubcoreMesh(core_axis_name="c", subcore_axis_name="s")
    out = jax.ShapeDtypeStruct((_E, _HH), jnp.uint32)
    f = functools.partial(
        pl.kernel,
        mesh=mesh,
        out_type=(out, out),
        scratch_types=[
            pltpu.VMEM((_EPT,), jnp.int32),
            pltpu.VMEM((_CH, _HH), jnp.uint32),
            pltpu.VMEM((_CH, _HH), jnp.uint32),
            pltpu.SemaphoreType.DMA,
            pltpu.SemaphoreType.DMA,
            pltpu.SemaphoreType.DMA,
            pltpu.SemaphoreType.DMA,
        ],
    )(_gather_body)
    return f(a, b, src, dst)


def _scatter_body(m0_hbm, m1_hbm, dst_hbm, zero_hbm, o0_hbm, o1_hbm,
                  idx0, idx1, idxt, buf0, buf1, buft, sl0, sl1, acc_sh):
    c = lax.axis_index("c")
    s = lax.axis_index("s")
    r0 = pl.multiple_of(s * _RPT, 8)
    pltpu.sync_copy(zero_hbm, acc_sh.at[pl.ds(r0, _RPT)])
    plsc.subcore_barrier()

    base = s * _EPT

    def run(m_hbm):
        def descs(i, bufs, sem, n=_CH):
            idxb, buf = bufs
            off = pl.multiple_of(base + i * _CH, 8)
            return (pltpu.make_async_copy(dst_hbm.at[pl.ds(off, n)],
                                          idxb, sem),
                    pltpu.make_async_copy(m_hbm.at[pl.ds(off, n)], buf, sem))

        class _Loads:
            def __init__(self, i, bufs, sem, n=_CH):
                self.d = descs(i, bufs, sem, n)

            def start(self):
                for d in self.d:
                    d.start()

            def wait(self):
                for d in self.d:
                    d.wait()

        class _Scat:
            def __init__(self, i, bufs, sem, n=_CH):
                self.bufs = bufs

            def start(self):
                idxb, buf = self.bufs
                pltpu.sync_copy(buf, acc_sh.at[idxb], add=True)

            def wait(self):
                pass

        _pipeline(_NCHF,
                  lambda i, bufs, sem, n=_CH: _Loads(i, bufs, sem, n),
                  lambda i, bufs, sem, n=_CH: _Scat(i, bufs, sem, n),
                  (idx0, buf0), sl0, sl0, (idx1, buf1), sl1, sl1)

        tb = (idxt, buft)
        _Loads(_NCHF, tb, sl0, _CHT).start()
        _Loads(_NCHF, tb, sl0, _CHT).wait()
        _Scat(_NCHF, tb, sl0, _CHT).start()

    @pl.when(c == 0)
    def _():
        run(m0_hbm)

    @pl.when(c == 1)
    def _():
        run(m1_hbm)

    plsc.subcore_barrier()

    @pl.when(jnp.logical_and(c == 0, s < _NT - 1))
    def _():
        pltpu.sync_copy(acc_sh.at[pl.ds(r0, _RPT)], o0_hbm.at[pl.ds(r0, _RPT)])

    @pl.when(jnp.logical_and(c == 1, s < _NT - 1))
    def _():
        pltpu.sync_copy(acc_sh.at[pl.ds(r0, _RPT)], o1_hbm.at[pl.ds(r0, _RPT)])

    @pl.when(jnp.logical_and(c == 0, s == _NT - 1))
    def _():
        pltpu.sync_copy(acc_sh.at[pl.ds(r0, _RPT_LAST)],
                        o0_hbm.at[pl.ds(r0, _RPT_LAST)])

    @pl.when(jnp.logical_and(c == 1, s == _NT - 1))
    def _():
        pltpu.sync_copy(acc_sh.at[pl.ds(r0, _RPT_LAST)],
                        o1_hbm.at[pl.ds(r0, _RPT_LAST)])


def _sc_scatter(m0, m1, dst, zeros_half):
    mesh = plsc.VectorSubcoreMesh(core_axis_name="c", subcore_axis_name="s")
    out = jax.ShapeDtypeStruct((_N, _HH), _F32)
    f = functools.partial(
        pl.kernel,
        mesh=mesh,
        out_type=(out, out),
        scratch_types=[
            pltpu.VMEM((_CH,), jnp.int32),
            pltpu.VMEM((_CH,), jnp.int32),
            pltpu.VMEM((_CHT,), jnp.int32),
            pltpu.VMEM((_CH, _HH), _F32),
            pltpu.VMEM((_CH, _HH), _F32),
            pltpu.VMEM((_CHT, _HH), _F32),
            pltpu.SemaphoreType.DMA,
            pltpu.SemaphoreType.DMA,
            pltpu.VMEM_SHARED((_ACC_N, _HH), _F32),
        ],
    )(_scatter_body)
    return f(m0, m1, dst, zeros_half)


def kernel(x, edge_index, edge_attr, u, batch, W_embed, b_embed, Wrel, brel,
           Wroot, We1, be1, We2, be2, Wn1, bn1, Wn2, bn2, Wp1, bp1, Wp2, bp2,
           Wp3, bp3):
    src = edge_index[0]
    dst = edge_index[1]

    x8 = jnp.pad(x, ((0, 0), (0, 1)))
    w8 = jnp.pad(W_embed, ((0, 1), (0, 0)))

    zeros_half = jnp.zeros((_RPT, _HH), _F32)
    ea16 = edge_attr.astype(jnp.bfloat16)
    wcats = [jnp.concatenate([We1[i][:_H], We1[i][_H:2 * _H], Wn1[i][:_H]],
                             axis=1) for i in range(4)]

    h, a, b, hw = _tc_embed_p(x8, w8, b_embed.reshape(1, _H), wcats[0])

    for i in range(4):
        asrc, bdst = _sc_gather(a, b, src, dst)
        m0, m1 = _tc_edge(asrc, bdst, ea16,
                          We1[i][2 * _H:].astype(jnp.bfloat16),
                          be1[i].reshape(1, _H), We2[i].astype(jnp.bfloat16),
                          be2[i].reshape(1, _H))
        g0, g1 = _sc_scatter(m0, m1, dst, zeros_half)
        nxt = _tc_node(h, hw, g0, g1, Wn1[i][_H:_H + _HH], Wn1[i][_H + _HH:],
                       bn1[i].reshape(1, _H), Wn2[i], bn2[i].reshape(1, _H),
                       wcats[i + 1] if i < 3 else None)
        if i < 3:
            h, a, b, hw = nxt
        else:
            h = nxt

    return _tc_final(h, u, Wp1[:_H], Wp1[_H:], bp1.reshape(1, _H),
                     Wp2, bp2.reshape(1, _HH), Wp3, bp3.reshape(1, 3))
